# baseline (speedup 1.0000x reference)
import functools

import jax
import jax.numpy as jnp
from jax import lax
from jax.experimental import pallas as pl
from jax.experimental.pallas import tpu as pltpu
from jax.experimental.pallas import tpu_sc as plsc

ROWS = 64
COLS = 32768
K = 8
NUM_CORES = 2
NUM_SUBCORES = 16
LANES = 16
NUM_WORKERS = NUM_CORES * NUM_SUBCORES
ROWS_PER_WORKER = ROWS // NUM_WORKERS
VREGS_PER_ROW = COLS // LANES
NLISTS = 4
UNROLL = 2


def _insert(tops, v):
    out = []
    for t in tops:
        hi = jnp.maximum(t, v)
        v = jnp.minimum(t, v)
        out.append(hi)
    return out


def _merge_lists(a, b, resort):
    m = [jnp.maximum(a[i], b[K - 1 - i]) for i in range(K)]
    if resort:
        for d in (4, 2, 1):
            nm = list(m)
            for i in range(K):
                if i & d == 0:
                    nm[i] = jnp.maximum(m[i], m[i + d])
                    nm[i + d] = jnp.minimum(m[i], m[i + d])
            m = nm
    return m


def _row_topk(buf):
    neg = jnp.full((LANES,), -jnp.inf, jnp.float32)
    step_v = LANES * NLISTS * UNROLL

    def step(i, carry):
        ls = [list(carry[g * K:(g + 1) * K]) for g in range(NLISTS)]
        base = i * step_v
        for u in range(UNROLL):
            for g in range(NLISTS):
                v = buf[0, pl.ds(base + (u * NLISTS + g) * LANES, LANES)]
                ls[g] = _insert(ls[g], v)
        return tuple(x for l in ls for x in l)

    carry = lax.fori_loop(0, VREGS_PER_ROW // (NLISTS * UNROLL), step,
                          (neg,) * (K * NLISTS))
    ls = [list(carry[g * K:(g + 1) * K]) for g in range(NLISTS)]

    r01 = _merge_lists(ls[0], ls[1], resort=True)
    r23 = _merge_lists(ls[2], ls[3], resort=True)
    tops = _merge_lists(r01, r23, resort=False)

    s = [jnp.sort(t) for t in tops]
    while len(s) > 1:
        nxt = []
        for i in range(0, len(s), 2):
            m = jnp.maximum(s[i], lax.rev(s[i + 1], (0,)))
            nxt.append(jnp.sort(m))
        s = nxt
    return lax.rev(s[0], (0,))


def _sc_topk(x_flat):
    mesh = plsc.VectorSubcoreMesh(core_axis_name="c", subcore_axis_name="s")

    @functools.partial(
        pl.kernel,
        mesh=mesh,
        out_type=jax.ShapeDtypeStruct((ROWS * K,), jnp.float32),
        scratch_types=[
            pltpu.VMEM((1, COLS), jnp.float32),
            pltpu.VMEM((1, COLS), jnp.float32),
            pltpu.VMEM((LANES + K,), jnp.float32),
            pltpu.SemaphoreType.DMA,
            pltpu.SemaphoreType.DMA,
        ],
        compiler_params=pltpu.CompilerParams(needs_layout_passes=False, use_tc_tiling_on_sc=True),
    )
    def k(x_hbm, out_hbm, buf0, buf1, outv, sem0, sem1):
        wid = lax.axis_index("s") * NUM_CORES + lax.axis_index("c")
        row0 = wid * ROWS_PER_WORKER
        cp0 = pltpu.async_copy(x_hbm.at[pl.ds(row0, 1)], buf0, sem0)
        cp1 = pltpu.async_copy(x_hbm.at[pl.ds(row0 + 1, 1)], buf1, sem1)
        cp0.wait()
        outv[pl.ds(0, LANES)] = _row_topk(buf0)
        cp1.wait()
        outv[pl.ds(K, LANES)] = _row_topk(buf1)
        pltpu.sync_copy(outv.at[pl.ds(0, 2 * K)],
                        out_hbm.at[pl.ds(row0 * K, 2 * K)])

    return k(x_flat)


def kernel(x):
    out = _sc_topk(x)
    return out.reshape(ROWS, K)


# speedup vs baseline: 1.0864x; 1.0864x over previous
import functools

import jax
import jax.numpy as jnp
from jax import lax
from jax.experimental import pallas as pl
from jax.experimental.pallas import tpu as pltpu
from jax.experimental.pallas import tpu_sc as plsc

ROWS = 64
COLS = 32768
K = 8
NUM_CORES = 2
NUM_SUBCORES = 16
LANES = 16
NUM_WORKERS = NUM_CORES * NUM_SUBCORES
ROWS_PER_WORKER = ROWS // NUM_WORKERS
VREGS_PER_ROW = COLS // LANES
CHUNK = 32
NCHUNKS = VREGS_PER_ROW // CHUNK


def _insert(tops, v):
    out = []
    for t in tops:
        hi = jnp.maximum(t, v)
        v = jnp.minimum(t, v)
        out.append(hi)
    return out


def _merge_lists(a, b):
    return [jnp.maximum(a[i], b[K - 1 - i]) for i in range(K)]


def _sort_tree_desc(vs):
    s = [jnp.sort(t) for t in vs]
    while len(s) > 1:
        s = [jnp.sort(jnp.maximum(s[i], lax.rev(s[i + 1], (0,))))
             for i in range(0, len(s), 2)]
    return lax.rev(s[0], (0,))


def _row_topk(buf, cms, cand):
    neg = jnp.full((LANES,), -jnp.inf, jnp.float32)

    def astep(i, c):
        base = i * CHUNK * LANES
        acc = [neg, neg, neg, neg]
        for u in range(CHUNK):
            v = buf[0, pl.ds(base + u * LANES, LANES)]
            acc[u % 4] = jnp.maximum(acc[u % 4], v)
        cm = jnp.maximum(jnp.maximum(acc[0], acc[1]),
                         jnp.maximum(acc[2], acc[3]))
        cms[pl.ds(i * LANES, LANES)] = cm
        return c

    lax.fori_loop(0, NCHUNKS, astep, 0)

    def bstep(i, carry):
        l0, l1 = list(carry[:K]), list(carry[K:])
        for u in range(4):
            v = cms[pl.ds((i * 4 + u) * LANES, LANES)]
            if u % 2 == 0:
                l0 = _insert(l0, v)
            else:
                l1 = _insert(l1, v)
        return tuple(l0) + tuple(l1)

    carry = lax.fori_loop(0, NCHUNKS // 4, bstep, (neg,) * (2 * K))
    sd = _sort_tree_desc(_merge_lists(list(carry[:K]), list(carry[K:])))
    t_vec = jnp.broadcast_to(sd[7], (LANES,))

    for j in range(2 * K):
        cand[pl.ds(j * LANES, LANES)] = neg

    def dstep(i, c):
        cm = cms[pl.ds(i * LANES, LANES)]
        n = plsc.all_reduce_population_count(cm >= t_vec)[0]

        @pl.when(n > 0)
        def _():
            l0 = [cand[pl.ds(j * LANES, LANES)] for j in range(K)]
            l1 = [cand[pl.ds((K + j) * LANES, LANES)] for j in range(K)]
            base = i * CHUNK * LANES
            for u in range(CHUNK):
                v = buf[0, pl.ds(base + u * LANES, LANES)]
                if u % 2 == 0:
                    l0 = _insert(l0, v)
                else:
                    l1 = _insert(l1, v)
            for j in range(K):
                cand[pl.ds(j * LANES, LANES)] = l0[j]
                cand[pl.ds((K + j) * LANES, LANES)] = l1[j]

        return c

    lax.fori_loop(0, NCHUNKS, dstep, 0)

    l0 = [cand[pl.ds(j * LANES, LANES)] for j in range(K)]
    l1 = [cand[pl.ds((K + j) * LANES, LANES)] for j in range(K)]
    return _sort_tree_desc(_merge_lists(l0, l1))


def _sc_topk(x):
    mesh = plsc.VectorSubcoreMesh(core_axis_name="c", subcore_axis_name="s")

    @functools.partial(
        pl.kernel,
        mesh=mesh,
        out_type=jax.ShapeDtypeStruct((ROWS * K,), jnp.float32),
        scratch_types=[
            pltpu.VMEM((1, COLS), jnp.float32),
            pltpu.VMEM((1, COLS), jnp.float32),
            pltpu.VMEM((NCHUNKS * LANES,), jnp.float32),
            pltpu.VMEM((2 * K * LANES,), jnp.float32),
            pltpu.VMEM((LANES + K,), jnp.float32),
            pltpu.SemaphoreType.DMA,
            pltpu.SemaphoreType.DMA,
        ],
        compiler_params=pltpu.CompilerParams(needs_layout_passes=False,
                                             use_tc_tiling_on_sc=True),
    )
    def k(x_hbm, out_hbm, buf0, buf1, cms, cand, outv, sem0, sem1):
        wid = lax.axis_index("s") * NUM_CORES + lax.axis_index("c")
        row0 = wid * ROWS_PER_WORKER
        cp0 = pltpu.async_copy(x_hbm.at[pl.ds(row0, 1)], buf0, sem0)
        cp1 = pltpu.async_copy(x_hbm.at[pl.ds(row0 + 1, 1)], buf1, sem1)
        cp0.wait()
        outv[pl.ds(0, LANES)] = _row_topk(buf0, cms, cand)
        cp1.wait()
        outv[pl.ds(K, LANES)] = _row_topk(buf1, cms, cand)
        pltpu.sync_copy(outv.at[pl.ds(0, 2 * K)],
                        out_hbm.at[pl.ds(row0 * K, 2 * K)])

    return k(x)


def kernel(x):
    out = _sc_topk(x)
    return out.reshape(ROWS, K)
